# Initial kernel scaffold; baseline (speedup 1.0000x reference)
import jax
import jax.numpy as jnp
from jax.experimental import pallas as pl


def kernel(pyolos, gboxes_ltrb, glabels):
    raise NotImplementedError("write your pallas kernel here")


# speedup vs baseline: 58.7214x; 58.7214x over previous
import jax
import jax.numpy as jnp
from jax.experimental import pallas as pl
from jax.experimental.pallas import tpu as pltpu

_NUM_CLASSES = 20
_NUM_ANC = 5
_GRID = 13
_BATCH = 16
_NBOX = 8
_S = 1 + _NUM_CLASSES
_NCELL = _GRID * _GRID
_NCH = (_S + 4) * _NUM_ANC
_EPS16 = 0.0009765625
_ANCW = (0.074, 0.147, 0.282, 0.471, 0.784)
_ANCH = (0.060, 0.151, 0.231, 0.425, 0.740)


def _loss_body(p_ref, pt_ref, gb_ref, gl_ref, out_ref, gat_ref):
    f32 = jnp.float32
    gb = gb_ref[...]
    gl = gl_ref[...]

    xy = (gb[..., :2] + gb[..., 2:]) * 0.5
    wh = gb[..., 2:] - gb[..., :2]
    w_g = wh[..., 0]
    h_g = wh[..., 1]
    aidx = jax.lax.broadcasted_iota(jnp.int32, (_BATCH, _NBOX, _NUM_ANC), 2)

    def _table(vals):
        t = jnp.full(aidx.shape, vals[-1], f32)
        for a in range(_NUM_ANC - 2, -1, -1):
            t = jnp.where(aidx == a, vals[a], t)
        return t

    ancw = _table(_ANCW)
    anch = _table(_ANCH)
    inter = jnp.minimum(w_g[..., None], ancw) * jnp.minimum(h_g[..., None], anch)
    area_g = (w_g * h_g)[..., None]
    iou = inter / (area_g + ancw * anch - inter)
    mx = jnp.max(iou, axis=-1, keepdims=True)
    ids = jnp.min(jnp.where(iou >= mx, aidx, _NUM_ANC), axis=-1)

    xys = xy * float(_GRID)
    fl = jnp.floor(xys)
    txy = xys - fl
    colrow = fl.astype(jnp.int32)
    cell = colrow[..., 1] * _GRID + colrow[..., 0]
    asel = (aidx == ids[..., None]).astype(f32)
    aw = jnp.sum(asel * ancw, axis=-1)
    ah = jnp.sum(asel * anch, axis=-1)
    twx = jnp.log(w_g / aw)
    twy = jnp.log(h_g / ah)
    wgt = 2.0 - w_g * h_g

    ii = jax.lax.broadcasted_iota(jnp.int32, (_BATCH, _NBOX, _NBOX), 1)
    jj = jax.lax.broadcasted_iota(jnp.int32, (_BATCH, _NBOX, _NBOX), 2)
    clob = (cell[:, None, :] == cell[:, :, None]) & (jj > ii)
    last = jnp.where(jnp.any(clob, axis=2), 0.0, 1.0)

    giota = jax.lax.broadcasted_iota(jnp.int32, (_BATCH, _NBOX, _NCELL), 2)
    oh = (giota == cell[..., None]).astype(f32)
    for b in range(_BATCH):
        gat_ref[b] = jax.lax.dot_general(
            oh[b], pt_ref[b], (((1,), (0,)), ((), ())),
            preferred_element_type=f32)
    gat = gat_ref[...]

    pcA = jnp.clip(jax.nn.sigmoid(p_ref[:, 0:_NUM_ANC, :]), 1e-7, 1.0 - 1e-7)
    fnegA = -0.5 * pcA * pcA * jnp.log(1.0 - pcA)
    neg_all = jnp.sum(jnp.sum(fnegA, axis=1), axis=1, keepdims=True)

    kiota = jax.lax.broadcasted_iota(jnp.int32, (_BATCH, _NBOX, _NCH), 2)
    cmap = jnp.floor(kiota.astype(f32) * 0.2).astype(jnp.int32)
    amap = kiota - cmap * _NUM_ANC
    am = amap == ids[..., None]
    pall = jax.nn.sigmoid(gat)
    pcl = jnp.clip(pall, 1e-7, 1.0 - 1e-7)
    logp = jnp.log(pcl)
    log1mp = jnp.log(1.0 - pcl)

    is_conf = cmap == 0
    fneg = -0.5 * pcl * pcl * log1mp
    sub = jnp.sum(jnp.where(is_conf, fneg, 0.0), axis=2)
    fpos = -0.5 * (1.0 - pcl) * (1.0 - pcl) * logp
    cpos = jnp.sum(jnp.where(is_conf & am, fpos, 0.0), axis=2)

    tcls = (cmap == gl[..., None]).astype(f32)
    bce_cls = -(tcls * logp + (1.0 - tcls) * log1mp)
    mcls = (cmap >= 1) & (cmap <= _NUM_CLASSES) & am
    clsv = jnp.sum(jnp.where(mcls, bce_cls, 0.0), axis=2)

    tgxy = jnp.where(cmap == _S, txy[..., 0:1], txy[..., 1:2])
    bce_xy = -(tgxy * logp + (1.0 - tgxy) * log1mp)
    mxy = ((cmap == _S) | (cmap == _S + 1)) & am
    xyv = jnp.sum(jnp.where(mxy, bce_xy, 0.0), axis=2)

    tgwh = jnp.where(cmap == _S + 2, twx[..., None], twy[..., None])
    dwh = gat - tgwh
    mwh = ((cmap == _S + 2) | (cmap == _S + 3)) & am
    whv = jnp.sum(jnp.where(mwh, dwh * dwh, 0.0), axis=2)

    npos_b = jnp.sum(last, axis=1, keepdims=True)
    npos_t = jnp.maximum(jnp.sum(npos_b), 1.0)
    nneg_b = jnp.maximum(float(_NCELL * _NUM_ANC) - 5.0 * npos_b, _EPS16)
    npos_bc = jnp.maximum(npos_b, _EPS16)
    neg_img = neg_all - jnp.sum(last * sub, axis=1, keepdims=True)
    l_conf_neg = jnp.sum(neg_img / nneg_b) * (3.0 / _BATCH)
    l_conf_pos = jnp.sum(jnp.sum(last * cpos, axis=1, keepdims=True) / npos_bc) / _BATCH
    l_cls = jnp.sum(last * clsv) / npos_t
    l_txty = jnp.sum(last * wgt * xyv) / npos_t
    l_twth = jnp.sum(last * wgt * whv) / npos_t

    total = l_conf_pos + l_conf_neg + l_cls + l_txty + l_twth
    out_ref[...] = jnp.broadcast_to(total, (1, 1))


def kernel(pyolos, gboxes_ltrb, glabels):
    p3 = pyolos.reshape(_BATCH, _NCH, _NCELL)
    pt = jnp.transpose(p3, (0, 2, 1))
    gl = glabels.astype(jnp.int32)
    out = pl.pallas_call(
        _loss_body,
        out_shape=jax.ShapeDtypeStruct((1, 1), jnp.float32),
        scratch_shapes=[pltpu.VMEM((_BATCH, _NBOX, _NCH), jnp.float32)],
    )(p3, pt, gboxes_ltrb.astype(jnp.float32), gl)
    return out[0, 0]
